# Initial kernel scaffold; baseline (speedup 1.0000x reference)
import jax
import jax.numpy as jnp
from jax.experimental import pallas as pl


def kernel(atomic_number, chirality_type, edge_index, bond_type, bond_direction_type, graph_ids, atom_emb, chir_emb, bond_embs, dir_embs, W1s, b1s, W2s, b2s, gate_w, gate_b):
    raise NotImplementedError("write your pallas kernel here")


# speedup vs baseline: 4.9036x; 4.9036x over previous
import functools

import numpy as np

import jax
import jax.numpy as jnp
from jax import lax
from jax.experimental import pallas as pl
from jax.experimental.pallas import tpu as pltpu
from jax.experimental.pallas import tpu_sc as plsc

_NC = 2
_NS = 16
_G = 64
_K = 80
_ZR = 104
_NPT = 624


def _row_chunks(s, N):
    chunks = [(s * _NPT + t * _ZR, _ZR, None) for t in range(_NPT // _ZR)]
    rem = N - 16 * _NPT
    if rem:
        chunks.append((16 * _NPT + 8 * s, 8, s < rem // 8))
    return chunks


_OHTAB = np.zeros((18, 128), np.float32)
for _b in range(6):
    for _d in range(3):
        _OHTAB[_b * 3 + _d, _b] = 1.0
        _OHTAB[_b * 3 + _d, 8 + _d] = 1.0


def _mesh():
    return plsc.VectorSubcoreMesh(core_axis_name="c", subcore_axis_name="s")


@functools.lru_cache(maxsize=None)
def _sc_embed(N, D):
    nchunks = N // _K
    per_w = nchunks // (_NC * _NS) + 1

    def body(ctab_h, cidx_h, h0_o, idx_v, msg_v, sem):
        c = lax.axis_index("c")
        s = lax.axis_index("s")
        w = c * _NS + s

        def chunk(t, carry):
            cid = w + t * (_NC * _NS)

            @pl.when(cid < nchunks)
            def _():
                base = cid * _K
                pltpu.sync_copy(cidx_h.at[pl.ds(base, _K)], idx_v)
                pltpu.async_copy(ctab_h.at[idx_v], msg_v, sem).wait()
                pltpu.sync_copy(msg_v, h0_o.at[pl.ds(base, _K)])
            return carry
        lax.fori_loop(0, per_w, chunk, None)

    return pl.kernel(
        body,
        out_type=jax.ShapeDtypeStruct((N, D), jnp.float32),
        mesh=_mesh(),
        scratch_types=[
            pltpu.VMEM((_K,), jnp.int32),
            pltpu.VMEM((_K, D), jnp.float32),
            pltpu.SemaphoreType.DMA,
        ],
    )


@functools.lru_cache(maxsize=None)
def _sc_layer0(N, E, D):
    EPT = E // _NS
    nchunks = EPT // _K

    def body(h0_h, src_h, dst_h, bt_h, bdt_h, ohtab_h, agg_o, cnt_o,
             src_v, dst_v, bt_v, bdt_v, et_v, msg_v, oh_v,
             zrow_v, acc_s, sem, sem2):
        c = lax.axis_index("c")
        s = lax.axis_index("s")

        zero16 = jnp.zeros((16,), jnp.float32)

        def _zr(t, carry):
            zrow_v[t // (D // 16), pl.ds((t % (D // 16)) * 16, 16)] = zero16
            return carry
        lax.fori_loop(0, _ZR * (D // 16), _zr, None)

        for off, nr, guard in _row_chunks(s, N):
            def _do(off=off, nr=nr):
                pltpu.sync_copy(zrow_v.at[pl.ds(0, nr)], acc_s.at[pl.ds(off, nr)])
            if guard is None:
                _do()
            else:
                pl.when(guard)(_do)
        plsc.subcore_barrier()

        ebase = s * EPT

        @pl.when(c == 0)
        def _agg_loop():
            def chunk(i, carry):
                base = ebase + i * _K
                pltpu.sync_copy(src_h.at[pl.ds(base, _K)], src_v)
                pltpu.sync_copy(dst_h.at[pl.ds(base, _K)], dst_v.at[0])
                pltpu.async_copy(h0_h.at[src_v], msg_v, sem).wait()
                pltpu.sync_copy(msg_v, acc_s.at[dst_v.at[0]], add=True)
                return carry
            lax.fori_loop(0, nchunks, chunk, None)

        @pl.when(c == 1)
        def _cnt_loop():
            def chunk(i, carry):
                base = ebase + i * _K
                pltpu.sync_copy(dst_h.at[pl.ds(base, _K)], dst_v.at[0])
                pltpu.sync_copy(bt_h.at[pl.ds(base, _K)], bt_v)
                pltpu.sync_copy(bdt_h.at[pl.ds(base, _K)], bdt_v)
                for g in range(_K // 16):
                    sl = pl.ds(g * 16, 16)
                    et_v[sl] = bt_v[sl] * 3 + bdt_v[sl]
                pltpu.async_copy(ohtab_h.at[et_v], oh_v, sem2).wait()
                pltpu.sync_copy(oh_v, acc_s.at[dst_v.at[0]], add=True)
                return carry
            lax.fori_loop(0, nchunks, chunk, None)

        plsc.subcore_barrier()

        for off, nr, guard in _row_chunks(s, N):
            def _out(off=off, nr=nr, guard=guard):
                g0 = c == 0 if guard is None else jnp.logical_and(c == 0, guard)
                g1 = c == 1 if guard is None else jnp.logical_and(c == 1, guard)

                def _o0():
                    pltpu.sync_copy(acc_s.at[pl.ds(off, nr)],
                                    agg_o.at[pl.ds(off, nr)])

                def _o1():
                    pltpu.sync_copy(acc_s.at[pl.ds(off, nr)],
                                    cnt_o.at[pl.ds(off, nr)])
                pl.when(g0)(_o0)
                pl.when(g1)(_o1)
            _out()

    return pl.kernel(
        body,
        out_type=(jax.ShapeDtypeStruct((N, D), jnp.float32),
                  jax.ShapeDtypeStruct((N, 128), jnp.float32)),
        mesh=_mesh(),
        scratch_types=[
            pltpu.VMEM((_K,), jnp.int32),
            pltpu.VMEM((1, _K), jnp.int32),
            pltpu.VMEM((_K,), jnp.int32),
            pltpu.VMEM((_K,), jnp.int32),
            pltpu.VMEM((_K,), jnp.int32),
            pltpu.VMEM((_K, D), jnp.float32),
            pltpu.VMEM((_K, 128), jnp.float32),
            pltpu.VMEM((_ZR, D), jnp.float32),
            pltpu.VMEM_SHARED((N, 128), jnp.float32),
            pltpu.SemaphoreType.DMA,
            pltpu.SemaphoreType.DMA,
        ],
    )


@functools.lru_cache(maxsize=None)
def _sc_gather(N, E, D):
    EPW = E // (_NC * _NS)
    nchunks = EPW // _K

    def body(h_h, src_h, dst_h, agg_o,
             src_v, dst_v, msg_v, zrow_v, agg_s, sem):
        c = lax.axis_index("c")
        s = lax.axis_index("s")
        w = c * _NS + s

        zero16 = jnp.zeros((16,), jnp.float32)

        def _zr(t, carry):
            zrow_v[t // (D // 16), pl.ds((t % (D // 16)) * 16, 16)] = zero16
            return carry
        lax.fori_loop(0, _ZR * (D // 16), _zr, None)

        for off, nr, guard in _row_chunks(s, N):
            def _do(off=off, nr=nr):
                pltpu.sync_copy(zrow_v.at[pl.ds(0, nr)], agg_s.at[pl.ds(off, nr)])
            if guard is None:
                _do()
            else:
                pl.when(guard)(_do)
        plsc.subcore_barrier()

        ebase = w * EPW

        def chunk(i, carry):
            base = ebase + i * _K
            pltpu.sync_copy(src_h.at[pl.ds(base, _K)], src_v)
            pltpu.sync_copy(dst_h.at[pl.ds(base, _K)], dst_v.at[0])
            pltpu.async_copy(h_h.at[src_v], msg_v, sem).wait()
            pltpu.sync_copy(msg_v, agg_s.at[dst_v.at[0]], add=True)
            return carry
        lax.fori_loop(0, nchunks, chunk, None)
        plsc.subcore_barrier()

        for off, nr, guard in _row_chunks(s, N):
            def _out(off=off, nr=nr):
                pltpu.sync_copy(agg_s.at[pl.ds(off, nr)], agg_o.at[c, pl.ds(off, nr)])
            if guard is None:
                _out()
            else:
                pl.when(guard)(_out)

    return pl.kernel(
        body,
        out_type=jax.ShapeDtypeStruct((_NC, N, D), jnp.float32),
        mesh=_mesh(),
        scratch_types=[
            pltpu.VMEM((_K,), jnp.int32),
            pltpu.VMEM((1, _K), jnp.int32),
            pltpu.VMEM((_K, D), jnp.float32),
            pltpu.VMEM((_ZR, D), jnp.float32),
            pltpu.VMEM_SHARED((N, D), jnp.float32),
            pltpu.SemaphoreType.DMA,
        ],
    )


def _prep(atom_emb, chir_emb, bond_embs, dir_embs, W1s):
    D = atom_emb.shape[1]
    L, _, H = W1s.shape
    NA = atom_emb.shape[0]
    NCH = chir_emb.shape[0]
    NT = NA * NCH

    def body(atom_ref, chir_ref, bond_ref, dir_ref, w1_ref, ctab_ref, emat_ref):
        row = lax.broadcasted_iota(jnp.int32, (NT, 1), 0)
        ra = ((row // NCH) == lax.broadcasted_iota(jnp.int32, (1, NA), 1)
              ).astype(jnp.float32)
        rc = ((row % NCH) == lax.broadcasted_iota(jnp.int32, (1, NCH), 1)
              ).astype(jnp.float32)
        ctab_ref[...] = (
            jnp.dot(ra, atom_ref[...], preferred_element_type=jnp.float32)
            + jnp.dot(rc, chir_ref[...], preferred_element_type=jnp.float32))
        i16 = lax.broadcasted_iota(jnp.int32, (16, 1), 0)
        mb = (i16 == lax.broadcasted_iota(jnp.int32, (1, 6), 1)).astype(jnp.float32)
        md = ((i16 - 8) == lax.broadcasted_iota(jnp.int32, (1, 3), 1)).astype(jnp.float32)
        for l in range(L):
            zb = jnp.dot(bond_ref[l], w1_ref[l], preferred_element_type=jnp.float32)
            zd = jnp.dot(dir_ref[l], w1_ref[l], preferred_element_type=jnp.float32)
            emat_ref[l] = (jnp.dot(mb, zb, preferred_element_type=jnp.float32)
                           + jnp.dot(md, zd, preferred_element_type=jnp.float32))

    return pl.pallas_call(
        body,
        out_shape=[jax.ShapeDtypeStruct((NT, D), jnp.float32),
                   jax.ShapeDtypeStruct((L, 16, H), jnp.float32)],
    )(atom_emb, chir_emb, bond_embs, dir_embs, W1s)


def _mlp(agg, cnt, W1, emat_l, b1r, W2, b2r, gate_w, gate_br, last):
    dual = agg.ndim == 3
    N, D = agg.shape[-2], agg.shape[-1]
    H = W1.shape[1]
    NB = 1000
    grid = N // NB

    def body(agg_ref, cnt_ref, w1_ref, em_ref, b1_ref, w2_ref, b2_ref,
             gw_ref, gb_ref, h_ref, *gate_out):
        if dual:
            a = agg_ref[0] + agg_ref[1]
        else:
            a = agg_ref[...]
        c16 = cnt_ref[:, :16]
        z = jnp.dot(a, w1_ref[...], preferred_element_type=jnp.float32)
        z = z + jnp.dot(c16, em_ref[...], preferred_element_type=jnp.float32)
        z = jnp.maximum(z + b1_ref[...], 0.0)
        h = jnp.dot(z, w2_ref[...], preferred_element_type=jnp.float32) + b2_ref[...]
        if not last:
            h = jnp.maximum(h, 0.0)
        h_ref[...] = h
        if last:
            gate_out[0][...] = (
                jnp.dot(h, gw_ref[...], preferred_element_type=jnp.float32)
                + gb_ref[...])

    agg_spec = (pl.BlockSpec((_NC, NB, D), lambda i: (0, i, 0)) if dual
                else pl.BlockSpec((NB, D), lambda i: (i, 0)))
    out_shape = [jax.ShapeDtypeStruct((N, D), jnp.float32)]
    out_specs = [pl.BlockSpec((NB, D), lambda i: (i, 0))]
    if last:
        out_shape.append(jax.ShapeDtypeStruct((N, 1), jnp.float32))
        out_specs.append(pl.BlockSpec((NB, 1), lambda i: (i, 0)))

    outs = pl.pallas_call(
        body,
        grid=(grid,),
        in_specs=[
            agg_spec,
            pl.BlockSpec((NB, 128), lambda i: (i, 0)),
            pl.BlockSpec((D, H), lambda i: (0, 0)),
            pl.BlockSpec((16, H), lambda i: (0, 0)),
            pl.BlockSpec((1, H), lambda i: (0, 0)),
            pl.BlockSpec((H, D), lambda i: (0, 0)),
            pl.BlockSpec((1, D), lambda i: (0, 0)),
            pl.BlockSpec((D, 1), lambda i: (0, 0)),
            pl.BlockSpec((1, 1), lambda i: (0, 0)),
        ],
        out_specs=out_specs,
        out_shape=out_shape,
    )(agg, cnt, W1, emat_l, b1r, W2, b2r, gate_w, gate_br)
    return outs if last else outs[0]


def _readout(h, gate, gids2d):
    N, D = h.shape

    def body(h_ref, g_ref, id_ref, o_ref):
        mask = id_ref[...] == lax.broadcasted_iota(jnp.int32, (1, _G), 1)
        maskf = mask.astype(jnp.float32)
        gate = g_ref[...]
        gb = jnp.where(mask, gate, -3e38)
        gmax = jnp.max(gb, axis=0, keepdims=True)
        gath = jnp.sum(maskf * gmax, axis=1, keepdims=True)
        eg = jnp.exp(gate - gath)
        denom = jnp.sum(maskf * eg, axis=0, keepdims=True)
        gden = jnp.sum(maskf * denom, axis=1, keepdims=True)
        wh = (eg / gden) * h_ref[...]
        o_ref[...] = lax.dot_general(
            maskf, wh, (((0,), (0,)), ((), ())),
            preferred_element_type=jnp.float32)

    return pl.pallas_call(
        body,
        out_shape=jax.ShapeDtypeStruct((_G, D), jnp.float32),
    )(h, gate, gids2d)


def kernel(atomic_number, chirality_type, edge_index, bond_type,
           bond_direction_type, graph_ids, atom_emb, chir_emb, bond_embs,
           dir_embs, W1s, b1s, W2s, b2s, gate_w, gate_b):
    N = atomic_number.shape[0]
    E = edge_index.shape[1]
    D = atom_emb.shape[1]
    L, _, H = W1s.shape

    src = edge_index[0].astype(jnp.int32)
    dst = edge_index[1].astype(jnp.int32)
    bt = bond_type.astype(jnp.int32)
    bdt = bond_direction_type.astype(jnp.int32)
    cidx = atomic_number.astype(jnp.int32) * 3 + chirality_type.astype(jnp.int32)

    ctab, emat = _prep(atom_emb, chir_emb, bond_embs, dir_embs, W1s)
    ohtab = jnp.asarray(_OHTAB)
    h0 = _sc_embed(N, D)(ctab, cidx)
    aggp, cntp = _sc_layer0(N, E, D)(h0, src, dst, bt, bdt, ohtab)
    del ctab

    b1r = b1s.reshape(L, 1, H)
    b2r = b2s.reshape(L, 1, D)
    gbr = gate_b.reshape(1, 1)

    h = None
    gate = None
    for l in range(L):
        if l > 0:
            aggp = _sc_gather(N, E, D)(h, src, dst)
        last = l == L - 1
        outs = _mlp(aggp, cntp, W1s[l], emat[l], b1r[l], W2s[l], b2r[l],
                    gate_w, gbr, last)
        if last:
            h, gate = outs
        else:
            h = outs

    gids2d = graph_ids.astype(jnp.int32).reshape(N, 1)
    return _readout(h, gate, gids2d)


# speedup vs baseline: 7.5371x; 1.5371x over previous
import functools

import numpy as np

import jax
import jax.numpy as jnp
from jax import lax
from jax.experimental import pallas as pl
from jax.experimental.pallas import tpu as pltpu
from jax.experimental.pallas import tpu_sc as plsc

_NC = 2
_NS = 16
_G = 64
_K = 80
_ZR = 48
_NPT = 624


_NB = 3
_PIPE_GATHER = True
_PIPE_L0 = True
_SYNC_SCATTER = False


def _edge_pipeline(nchunks, fire_idx, fire_gather, fire_scatter, wait_scatter):
    J = nchunks // _NB
    assert J >= 3
    fire_idx(0, 0)
    fire_idx(1, 1)
    fire_gather(0, 0)
    for i in range(_NB):
        if i >= 1:
            wait_scatter(i - 1, (i - 1) % _NB)
        fire_idx(i + 2, (i + 2) % _NB)
        fire_gather(i + 1, (i + 1) % _NB)
        fire_scatter(i, i % _NB)

    def body(j, carry):
        i0 = j * _NB
        for b in range(_NB):
            i = i0 + b
            wait_scatter(i - 1, (b + _NB - 1) % _NB)
            fire_idx(i + 2, (b + 2) % _NB)
            fire_gather(i + 1, (b + 1) % _NB)
            fire_scatter(i, b)
        return carry
    lax.fori_loop(1, J - 1, body, None)

    for i in range((J - 1) * _NB, nchunks):
        wait_scatter(i - 1, (i - 1) % _NB)
        if i + 2 < nchunks:
            fire_idx(i + 2, (i + 2) % _NB)
        if i + 1 < nchunks:
            fire_gather(i + 1, (i + 1) % _NB)
        fire_scatter(i, i % _NB)
    wait_scatter(nchunks - 1, (nchunks - 1) % _NB)


def _row_chunks(s, N):
    chunks = [(s * _NPT + t * _ZR, _ZR, None) for t in range(_NPT // _ZR)]
    rem = N - 16 * _NPT
    if rem:
        chunks.append((16 * _NPT + 8 * s, 8, s < rem // 8))
    return chunks


_OHTAB = np.zeros((18, 128), np.float32)
for _b in range(6):
    for _d in range(3):
        _OHTAB[_b * 3 + _d, _b] = 1.0
        _OHTAB[_b * 3 + _d, 8 + _d] = 1.0


def _mesh():
    return plsc.VectorSubcoreMesh(core_axis_name="c", subcore_axis_name="s")


@functools.lru_cache(maxsize=None)
def _sc_embed(N, D):
    nchunks = N // _K
    per_w = (nchunks + _NC * _NS - 1) // (_NC * _NS)

    def body(ctab_h, cidx_h, h0_o, idx_v, msg_v, sem):
        c = lax.axis_index("c")
        s = lax.axis_index("s")
        w = c * _NS + s

        def chunk(t, carry):
            cid = w + t * (_NC * _NS)

            @pl.when(cid < nchunks)
            def _():
                base = cid * _K
                pltpu.sync_copy(cidx_h.at[pl.ds(base, _K)], idx_v)
                pltpu.async_copy(ctab_h.at[idx_v], msg_v, sem).wait()
                pltpu.sync_copy(msg_v, h0_o.at[pl.ds(base, _K)])
            return carry
        lax.fori_loop(0, per_w, chunk, None)

    return pl.kernel(
        body,
        out_type=jax.ShapeDtypeStruct((N, D), jnp.float32),
        mesh=_mesh(),
        scratch_types=[
            pltpu.VMEM((_K,), jnp.int32),
            pltpu.VMEM((_K, D), jnp.float32),
            pltpu.SemaphoreType.DMA,
        ],
    )


@functools.lru_cache(maxsize=None)
def _sc_layer0(N, E, D):
    EPT = E // _NS
    nchunks = EPT // _K

    def body(h0_h, src_h, dst_h, bt_h, bdt_h, ohtab_h,
             agg_o, cnt_o,
             src_i, dst_i, bt_i, bdt_i, et_i, *rest):
        msgs = rest[:_NB]
        zrow_v = rest[_NB]
        acc_s = rest[_NB + 1]
        sems = rest[_NB + 2:]
        isems = sems[0:_NB]
        gsems = sems[_NB:2 * _NB]
        ssems = sems[2 * _NB:3 * _NB]

        c = lax.axis_index("c")
        s = lax.axis_index("s")

        zero16 = jnp.zeros((16,), jnp.float32)

        def _zr(t, carry):
            zrow_v[t // (D // 16), pl.ds((t % (D // 16)) * 16, 16)] = zero16
            return carry
        lax.fori_loop(0, _ZR * (D // 16), _zr, None)

        for off, nr, guard in _row_chunks(s, N):
            def _do(off=off, nr=nr):
                pltpu.sync_copy(zrow_v.at[pl.ds(0, nr)], acc_s.at[pl.ds(off, nr)])
            if guard is None:
                _do()
            else:
                pl.when(guard)(_do)
        plsc.subcore_barrier()

        ebase = s * EPT

        @pl.when(c == 0)
        def _agg_loop():
            def fire_idx(i, b):
                base = ebase + i * _K
                pltpu.async_copy(src_h.at[pl.ds(base, _K)], src_i.at[b], isems[b])
                pltpu.async_copy(dst_h.at[pl.ds(base, _K)], dst_i.at[b], isems[b])

            def fire_gather(i, b):
                base = ebase + i * _K
                pltpu.make_async_copy(src_h.at[pl.ds(base, _K)], src_i.at[b],
                                      isems[b]).wait()
                pltpu.make_async_copy(dst_h.at[pl.ds(base, _K)], dst_i.at[b],
                                      isems[b]).wait()
                pltpu.async_copy(h0_h.at[src_i.at[b]], msgs[b], gsems[b])

            def fire_scatter(i, b):
                base = ebase + i * _K
                pltpu.make_async_copy(h0_h.at[src_i.at[b]], msgs[b],
                                      gsems[b]).wait()
                if _SYNC_SCATTER:
                    pltpu.sync_copy(msgs[b], acc_s.at[dst_i.at[b]], add=True)
                else:
                    pltpu.async_copy(msgs[b], acc_s.at[dst_i.at[b]], ssems[b],
                                     add=True)

            def wait_scatter(i, b):
                if not _SYNC_SCATTER:
                    pltpu.make_async_copy(msgs[b], acc_s.at[dst_i.at[b]],
                                          ssems[b]).wait()

            if _PIPE_L0:
                _edge_pipeline(nchunks, fire_idx, fire_gather, fire_scatter,
                               wait_scatter)
            else:
                def chunk(i, carry):
                    base = ebase + i * _K
                    pltpu.sync_copy(src_h.at[pl.ds(base, _K)], src_i.at[0])
                    pltpu.sync_copy(dst_h.at[pl.ds(base, _K)], dst_i.at[0])
                    pltpu.async_copy(h0_h.at[src_i.at[0]], msgs[0],
                                     gsems[0]).wait()
                    pltpu.sync_copy(msgs[0], acc_s.at[dst_i.at[0]], add=True)
                    return carry
                lax.fori_loop(0, nchunks, chunk, None)

        @pl.when(c == 1)
        def _cnt_loop():
            def fire_idx(i, b):
                base = ebase + i * _K
                pltpu.async_copy(dst_h.at[pl.ds(base, _K)], dst_i.at[b], isems[b])
                pltpu.async_copy(bt_h.at[pl.ds(base, _K)], bt_i.at[b], isems[b])
                pltpu.async_copy(bdt_h.at[pl.ds(base, _K)], bdt_i.at[b], isems[b])

            def fire_gather(i, b):
                base = ebase + i * _K
                pltpu.make_async_copy(dst_h.at[pl.ds(base, _K)], dst_i.at[b],
                                      isems[b]).wait()
                pltpu.make_async_copy(bt_h.at[pl.ds(base, _K)], bt_i.at[b],
                                      isems[b]).wait()
                pltpu.make_async_copy(bdt_h.at[pl.ds(base, _K)], bdt_i.at[b],
                                      isems[b]).wait()
                for g in range(_K // 16):
                    sl = pl.ds(g * 16, 16)
                    et_i[b, sl] = bt_i[b, sl] * 3 + bdt_i[b, sl]
                pltpu.async_copy(ohtab_h.at[et_i.at[b]], msgs[b], gsems[b])

            def fire_scatter(i, b):
                base = ebase + i * _K
                pltpu.make_async_copy(ohtab_h.at[et_i.at[b]], msgs[b],
                                      gsems[b]).wait()
                if _SYNC_SCATTER:
                    pltpu.sync_copy(msgs[b], acc_s.at[dst_i.at[b]], add=True)
                else:
                    pltpu.async_copy(msgs[b], acc_s.at[dst_i.at[b]], ssems[b],
                                     add=True)

            def wait_scatter(i, b):
                if not _SYNC_SCATTER:
                    pltpu.make_async_copy(msgs[b], acc_s.at[dst_i.at[b]],
                                          ssems[b]).wait()

            if _PIPE_L0:
                _edge_pipeline(nchunks, fire_idx, fire_gather, fire_scatter,
                               wait_scatter)
            else:
                def chunk(i, carry):
                    base = ebase + i * _K
                    pltpu.sync_copy(dst_h.at[pl.ds(base, _K)], dst_i.at[0])
                    pltpu.sync_copy(bt_h.at[pl.ds(base, _K)], bt_i.at[0])
                    pltpu.sync_copy(bdt_h.at[pl.ds(base, _K)], bdt_i.at[0])
                    for g in range(_K // 16):
                        sl = pl.ds(g * 16, 16)
                        et_i[0, sl] = bt_i[0, sl] * 3 + bdt_i[0, sl]
                    pltpu.async_copy(ohtab_h.at[et_i.at[0]], msgs[0],
                                     gsems[0]).wait()
                    pltpu.sync_copy(msgs[0], acc_s.at[dst_i.at[0]], add=True)
                    return carry
                lax.fori_loop(0, nchunks, chunk, None)

        plsc.subcore_barrier()

        for off, nr, guard in _row_chunks(s, N):
            def _out(off=off, nr=nr, guard=guard):
                g0 = c == 0 if guard is None else jnp.logical_and(c == 0, guard)
                g1 = c == 1 if guard is None else jnp.logical_and(c == 1, guard)

                def _o0():
                    pltpu.sync_copy(acc_s.at[pl.ds(off, nr)],
                                    agg_o.at[pl.ds(off, nr)])

                def _o1():
                    pltpu.sync_copy(acc_s.at[pl.ds(off, nr)],
                                    cnt_o.at[pl.ds(off, nr)])
                pl.when(g0)(_o0)
                pl.when(g1)(_o1)
            _out()

    return pl.kernel(
        body,
        out_type=(jax.ShapeDtypeStruct((N, D), jnp.float32),
                  jax.ShapeDtypeStruct((N, 128), jnp.float32)),
        mesh=_mesh(),
        scratch_types=(
            [pltpu.VMEM((_NB, _K), jnp.int32)] * 5
            + [pltpu.VMEM((_K, 128), jnp.float32)] * _NB
            + [pltpu.VMEM((_ZR, D), jnp.float32),
               pltpu.VMEM_SHARED((N, 128), jnp.float32)]
            + [pltpu.SemaphoreType.DMA] * (3 * _NB)
        ),
    )


@functools.lru_cache(maxsize=None)
def _sc_gather(N, E, D):
    EPW = E // (_NC * _NS)
    nchunks = EPW // _K

    def body(h_h, src_h, dst_h, agg_o, src_i, dst_i, *rest):
        msgs = rest[:_NB]
        zrow_v = rest[_NB]
        agg_s = rest[_NB + 1]
        sems = rest[_NB + 2:]
        isems = sems[0:_NB]
        gsems = sems[_NB:2 * _NB]
        ssems = sems[2 * _NB:3 * _NB]

        c = lax.axis_index("c")
        s = lax.axis_index("s")
        w = c * _NS + s

        zero16 = jnp.zeros((16,), jnp.float32)

        def _zr(t, carry):
            zrow_v[t // (D // 16), pl.ds((t % (D // 16)) * 16, 16)] = zero16
            return carry
        lax.fori_loop(0, _ZR * (D // 16), _zr, None)

        for off, nr, guard in _row_chunks(s, N):
            def _do(off=off, nr=nr):
                pltpu.sync_copy(zrow_v.at[pl.ds(0, nr)], agg_s.at[pl.ds(off, nr)])
            if guard is None:
                _do()
            else:
                pl.when(guard)(_do)
        plsc.subcore_barrier()

        ebase = w * EPW

        def fire_idx(i, b):
            base = ebase + i * _K
            pltpu.async_copy(src_h.at[pl.ds(base, _K)], src_i.at[b], isems[b])
            pltpu.async_copy(dst_h.at[pl.ds(base, _K)], dst_i.at[b], isems[b])

        def fire_gather(i, b):
            base = ebase + i * _K
            pltpu.make_async_copy(src_h.at[pl.ds(base, _K)], src_i.at[b],
                                  isems[b]).wait()
            pltpu.make_async_copy(dst_h.at[pl.ds(base, _K)], dst_i.at[b],
                                  isems[b]).wait()
            pltpu.async_copy(h_h.at[src_i.at[b]], msgs[b], gsems[b])

        def fire_scatter(i, b):
            base = ebase + i * _K
            pltpu.make_async_copy(h_h.at[src_i.at[b]], msgs[b], gsems[b]).wait()
            if _SYNC_SCATTER:
                pltpu.sync_copy(msgs[b], agg_s.at[dst_i.at[b]], add=True)
            else:
                pltpu.async_copy(msgs[b], agg_s.at[dst_i.at[b]], ssems[b],
                                 add=True)

        def wait_scatter(i, b):
            if not _SYNC_SCATTER:
                pltpu.make_async_copy(msgs[b], agg_s.at[dst_i.at[b]],
                                      ssems[b]).wait()

        if _PIPE_GATHER:
            _edge_pipeline(nchunks, fire_idx, fire_gather, fire_scatter,
                           wait_scatter)
        else:
            def chunk(i, carry):
                base = ebase + i * _K
                pltpu.sync_copy(src_h.at[pl.ds(base, _K)], src_i.at[0])
                pltpu.sync_copy(dst_h.at[pl.ds(base, _K)], dst_i.at[0])
                pltpu.async_copy(h_h.at[src_i.at[0]], msgs[0], gsems[0]).wait()
                pltpu.sync_copy(msgs[0], agg_s.at[dst_i.at[0]], add=True)
                return carry
            lax.fori_loop(0, nchunks, chunk, None)
        plsc.subcore_barrier()

        for off, nr, guard in _row_chunks(s, N):
            def _out(off=off, nr=nr):
                pltpu.sync_copy(agg_s.at[pl.ds(off, nr)], agg_o.at[c, pl.ds(off, nr)])
            if guard is None:
                _out()
            else:
                pl.when(guard)(_out)

    return pl.kernel(
        body,
        out_type=jax.ShapeDtypeStruct((_NC, N, D), jnp.float32),
        mesh=_mesh(),
        scratch_types=(
            [pltpu.VMEM((_NB, _K), jnp.int32),
             pltpu.VMEM((_NB, _K), jnp.int32)]
            + [pltpu.VMEM((_K, D), jnp.float32)] * _NB
            + [pltpu.VMEM((_ZR, D), jnp.float32),
               pltpu.VMEM_SHARED((N, D), jnp.float32)]
            + [pltpu.SemaphoreType.DMA] * (3 * _NB)
        ),
    )


def _prep(atom_emb, chir_emb, bond_embs, dir_embs, W1s):
    D = atom_emb.shape[1]
    L, _, H = W1s.shape
    NA = atom_emb.shape[0]
    NCH = chir_emb.shape[0]
    NT = NA * NCH

    def body(atom_ref, chir_ref, bond_ref, dir_ref, w1_ref, ctab_ref, emat_ref):
        row = lax.broadcasted_iota(jnp.int32, (NT, 1), 0)
        ra = ((row // NCH) == lax.broadcasted_iota(jnp.int32, (1, NA), 1)
              ).astype(jnp.float32)
        rc = ((row % NCH) == lax.broadcasted_iota(jnp.int32, (1, NCH), 1)
              ).astype(jnp.float32)
        ctab_ref[...] = (
            jnp.dot(ra, atom_ref[...], preferred_element_type=jnp.float32)
            + jnp.dot(rc, chir_ref[...], preferred_element_type=jnp.float32))
        i16 = lax.broadcasted_iota(jnp.int32, (16, 1), 0)
        mb = (i16 == lax.broadcasted_iota(jnp.int32, (1, 6), 1)).astype(jnp.float32)
        md = ((i16 - 8) == lax.broadcasted_iota(jnp.int32, (1, 3), 1)).astype(jnp.float32)
        for l in range(L):
            zb = jnp.dot(bond_ref[l], w1_ref[l], preferred_element_type=jnp.float32)
            zd = jnp.dot(dir_ref[l], w1_ref[l], preferred_element_type=jnp.float32)
            emat_ref[l] = (jnp.dot(mb, zb, preferred_element_type=jnp.float32)
                           + jnp.dot(md, zd, preferred_element_type=jnp.float32))

    return pl.pallas_call(
        body,
        out_shape=[jax.ShapeDtypeStruct((NT, D), jnp.float32),
                   jax.ShapeDtypeStruct((L, 16, H), jnp.float32)],
    )(atom_emb, chir_emb, bond_embs, dir_embs, W1s)


def _mlp(agg, cnt, W1, emat_l, b1r, W2, b2r, gate_w, gate_br, last,
         dummy=None):
    dual = agg.ndim == 3
    N, D = agg.shape[-2], agg.shape[-1]
    H = W1.shape[1]
    NB = 1000
    grid = N // NB

    def body(agg_ref, cnt_ref, w1_ref, em_ref, b1_ref, w2_ref, b2_ref,
             gw_ref, gb_ref, *out_refs):
        if dummy is not None:
            out_refs = out_refs[1:]
        h_ref = out_refs[0]
        gate_out = out_refs[1:]
        if dual:
            a = agg_ref[0] + agg_ref[1]
        else:
            a = agg_ref[...]
        c16 = cnt_ref[:, :16]
        z = jnp.dot(a, w1_ref[...], preferred_element_type=jnp.float32)
        z = z + jnp.dot(c16, em_ref[...], preferred_element_type=jnp.float32)
        z = jnp.maximum(z + b1_ref[...], 0.0)
        h = jnp.dot(z, w2_ref[...], preferred_element_type=jnp.float32) + b2_ref[...]
        if not last:
            h = jnp.maximum(h, 0.0)
        h_ref[...] = h
        if last:
            gate_out[0][...] = (
                jnp.dot(h, gw_ref[...], preferred_element_type=jnp.float32)
                + gb_ref[...])

    agg_spec = (pl.BlockSpec((_NC, NB, D), lambda i: (0, i, 0)) if dual
                else pl.BlockSpec((NB, D), lambda i: (i, 0)))
    out_shape = [jax.ShapeDtypeStruct((N, D), jnp.float32)]
    out_specs = [pl.BlockSpec((NB, D), lambda i: (i, 0))]
    if last:
        out_shape.append(jax.ShapeDtypeStruct((N, 1), jnp.float32))
        out_specs.append(pl.BlockSpec((NB, 1), lambda i: (i, 0)))

    outs = pl.pallas_call(
        body,
        grid=(grid,),
        in_specs=[
            agg_spec,
            pl.BlockSpec((NB, 128), lambda i: (i, 0)),
            pl.BlockSpec((D, H), lambda i: (0, 0)),
            pl.BlockSpec((16, H), lambda i: (0, 0)),
            pl.BlockSpec((1, H), lambda i: (0, 0)),
            pl.BlockSpec((H, D), lambda i: (0, 0)),
            pl.BlockSpec((1, D), lambda i: (0, 0)),
            pl.BlockSpec((D, 1), lambda i: (0, 0)),
            pl.BlockSpec((1, 1), lambda i: (0, 0)),
        ] + ([pl.BlockSpec((8, D), lambda i: (0, 0))] if dummy is not None else []),
        out_specs=out_specs,
        out_shape=out_shape,
    )(agg, cnt, W1, emat_l, b1r, W2, b2r, gate_w, gate_br,
      *(() if dummy is None else (dummy,)))
    return outs if last else outs[0]


def _readout(h, gate, gids2d):
    N, D = h.shape

    def body(h_ref, g_ref, id_ref, o_ref):
        mask = id_ref[...] == lax.broadcasted_iota(jnp.int32, (1, _G), 1)
        maskf = mask.astype(jnp.float32)
        gate = g_ref[...]
        gb = jnp.where(mask, gate, -3e38)
        gmax = jnp.max(gb, axis=0, keepdims=True)
        gath = jnp.sum(maskf * gmax, axis=1, keepdims=True)
        eg = jnp.exp(gate - gath)
        denom = jnp.sum(maskf * eg, axis=0, keepdims=True)
        gden = jnp.sum(maskf * denom, axis=1, keepdims=True)
        wh = (eg / gden) * h_ref[...]
        o_ref[...] = lax.dot_general(
            maskf, wh, (((0,), (0,)), ((), ())),
            preferred_element_type=jnp.float32)

    return pl.pallas_call(
        body,
        out_shape=jax.ShapeDtypeStruct((_G, D), jnp.float32),
    )(h, gate, gids2d)


def kernel(atomic_number, chirality_type, edge_index, bond_type,
           bond_direction_type, graph_ids, atom_emb, chir_emb, bond_embs,
           dir_embs, W1s, b1s, W2s, b2s, gate_w, gate_b):
    N = atomic_number.shape[0]
    E = edge_index.shape[1]
    D = atom_emb.shape[1]
    L, _, H = W1s.shape

    src = edge_index[0].astype(jnp.int32)
    dst = edge_index[1].astype(jnp.int32)
    bt = bond_type.astype(jnp.int32)
    bdt = bond_direction_type.astype(jnp.int32)
    cidx = atomic_number.astype(jnp.int32) * 3 + chirality_type.astype(jnp.int32)

    ctab, emat = _prep(atom_emb, chir_emb, bond_embs, dir_embs, W1s)
    ohtab = jnp.asarray(_OHTAB)
    h0 = _sc_embed(N, D)(ctab, cidx)
    aggp, cntp = _sc_layer0(N, E, D)(h0, src, dst, bt, bdt, ohtab)

    b1r = b1s.reshape(L, 1, H)
    b2r = b2s.reshape(L, 1, D)
    gbr = gate_b.reshape(1, 1)

    h = None
    gate = None
    for l in range(L):
        if l > 0:
            aggp = _sc_gather(N, E, D)(h, src, dst)
        last = l == L - 1
        outs = _mlp(aggp, cntp, W1s[l], emat[l], b1r[l], W2s[l], b2r[l],
                    gate_w, gbr, last)
        if last:
            h, gate = outs
        else:
            h = outs

    gids2d = graph_ids.astype(jnp.int32).reshape(N, 1)
    return _readout(h, gate, gids2d)
